# baseline (speedup 1.0000x reference)
import functools

import jax
import jax.numpy as jnp
from jax import lax
from jax.experimental import pallas as pl
from jax.experimental.pallas import tpu as pltpu
from jax.experimental.pallas import tpu_sc as plsc

_N_ROWS = 128
_N_COLS = 32768
_N_WORKERS = 32
_GROUP_ROWS = 8
_HALF_COLS = _N_COLS // 2
_CHUNK_COLS = 2048
_N_CHUNKS = _HALF_COLS // _CHUNK_COLS
_VECS_PER_BLOCK = 16
_BLOCK_COLS = 16 * _VECS_PER_BLOCK
_BLOCKS_PER_CHUNK = _CHUNK_COLS // _BLOCK_COLS


def _insert(v, n, t1, t2, t3, x1, x2, x3):
    c1 = v > t1
    c2 = v > t2
    c3 = v > t3
    nt1 = jnp.where(c1, v, t1)
    nx1 = jnp.where(c1, n, x1)
    nt2 = jnp.where(c1, t1, jnp.where(c2, v, t2))
    nx2 = jnp.where(c1, x1, jnp.where(c2, n, x2))
    nt3 = jnp.where(c2, t2, jnp.where(c3, v, t3))
    nx3 = jnp.where(c2, x2, jnp.where(c3, n, x3))
    return nt1, nt2, nt3, nx1, nx2, nx3


def _scan_chunk_row(buf, stv, sti, r, chunk_vec0):
    states = [_load_state(stv, sti, r, 48 * j) for j in range(4)]
    n0 = jnp.full((16,), 0, jnp.int32) + chunk_vec0
    ns = tuple(n0 + j for j in range(4))

    def body(i, carry):
        s = [carry[6 * j:6 * j + 6] for j in range(4)]
        n = list(carry[24:28])
        base = i * 64
        for j in range(4):
            v = buf[r, pl.ds(base + j * 16, 16)]
            s[j] = _insert(v, n[j], *s[j])
            n[j] = n[j] + 4
        return s[0] + s[1] + s[2] + s[3] + tuple(n)

    out = lax.fori_loop(0, _CHUNK_COLS // (16 * 4), body,
                        states[0] + states[1] + states[2] + states[3] + ns)
    for j in range(4):
        _store_state(stv, sti, r, out[6 * j:6 * j + 6], 48 * j)


def _load_state(stv, sti, r, off):
    return (stv[r, pl.ds(off, 16)], stv[r, pl.ds(off + 16, 16)],
            stv[r, pl.ds(off + 32, 16)],
            sti[r, pl.ds(off, 16)], sti[r, pl.ds(off + 16, 16)],
            sti[r, pl.ds(off + 32, 16)])


def _store_state(stv, sti, r, s, off):
    for k in range(3):
        stv[r, pl.ds(off + 16 * k, 16)] = s[k]
        sti[r, pl.ds(off + 16 * k, 16)] = s[3 + k]


def _sc_topk_body(scores_hbm, vals_hbm, idx_hbm, buf_a, buf_b, stv, sti,
                  rv, ri, sem_a, sem_b):
    wid = lax.axis_index("s") * 2 + lax.axis_index("c")
    g = wid // 2
    h = wid % 2
    row0 = g * _GROUP_ROWS
    col0 = h * _HALF_COLS

    neg = jnp.full((16,), -jnp.inf, jnp.float32)
    zero = jnp.zeros((16,), jnp.int32)
    for r in range(_GROUP_ROWS):
        for j in range(4):
            _store_state(stv, sti, r, (neg, neg, neg, zero, zero, zero), 48 * j)

    def chunk_src(c):
        start = pl.multiple_of(col0 + c * _CHUNK_COLS, _CHUNK_COLS)
        return scores_hbm.at[pl.ds(row0, _GROUP_ROWS),
                             pl.ds(start, _CHUNK_COLS)]

    def scan_buf(buf, c):
        chunk_vec0 = c * (_CHUNK_COLS // 16)
        for r in range(_GROUP_ROWS):
            _scan_chunk_row(buf, stv, sti, r, chunk_vec0)

    last = _N_CHUNKS - 1
    pltpu.async_copy(chunk_src(0), buf_a, sem_a).wait()

    def pair(p, carry):
        c = p * 2
        cp_b = pltpu.async_copy(chunk_src(jnp.minimum(c + 1, last)), buf_b, sem_b)
        scan_buf(buf_a, c)
        cp_b.wait()
        cp_a = pltpu.async_copy(chunk_src(jnp.minimum(c + 2, last)), buf_a, sem_a)
        scan_buf(buf_b, c + 1)
        cp_a.wait()
        return carry

    lax.fori_loop(0, _N_CHUNKS // 2, pair, 0)

    lane = lax.broadcasted_iota(jnp.int32, (16,), 0)
    big = 1 << 30

    def lex(av, an, bv, bn):
        return (av > bv) | ((av == bv) & (an < bn))

    def psel(cond, x, y):
        return (jnp.where(cond, x[0], y[0]), jnp.where(cond, x[1], y[1]))

    def pair_merge(a, b):
        ah, am, al = (a[0], a[3]), (a[1], a[4]), (a[2], a[5])
        bh, bm, bl = (b[0], b[3]), (b[1], b[4]), (b[2], b[5])
        merged = []
        for _k in range(3):
            ge = lex(ah[0], ah[1], bh[0], bh[1])
            merged.append(psel(ge, ah, bh))
            ah, am, al = psel(ge, am, ah), psel(ge, al, am), al
            bh, bm, bl = psel(~ge, bm, bh), psel(~ge, bl, bm), bl
        return (merged[0][0], merged[1][0], merged[2][0],
                merged[0][1], merged[1][1], merged[2][1])

    for r in range(_GROUP_ROWS):
        s0 = _load_state(stv, sti, r, 0)
        s1 = _load_state(stv, sti, r, 48)
        s2 = _load_state(stv, sti, r, 96)
        s3 = _load_state(stv, sti, r, 144)
        t1, t2, t3, x1, x2, x3 = pair_merge(pair_merge(s0, s1),
                                            pair_merge(s2, s3))
        g1 = x1 * 16 + lane + col0
        g2 = x2 * 16 + lane + col0
        g3 = x3 * 16 + lane + col0
        ms = []
        gs = []
        for _round in range(3):
            mx = jnp.max(t1)
            gi = jnp.min(jnp.where(t1 == mx, g1, big))
            win = g1 == gi
            ms.append(mx)
            gs.append(gi)
            t1 = jnp.where(win, t2, t1)
            g1 = jnp.where(win, g2, g1)
            t2 = jnp.where(win, t3, t2)
            g2 = jnp.where(win, g3, g2)
            t3 = jnp.where(win, -jnp.inf, t3)
        l0 = lane == 0
        l1 = lane == 1
        l2 = lane == 2
        valv = jnp.where(l0, ms[0],
                         jnp.where(l1, ms[1],
                                   jnp.where(l2, ms[2], jnp.float32(0.0))))
        idxv = jnp.where(l0, gs[0],
                         jnp.where(l1, gs[1], jnp.where(l2, gs[2], 0)))
        rv[pl.ds(16 * r, 16)] = valv
        ri[pl.ds(16 * r, 16)] = idxv
    pltpu.sync_copy(rv, vals_hbm.at[wid])
    pltpu.sync_copy(ri, idx_hbm.at[wid])


def _sc_topk(scores):
    mesh = plsc.VectorSubcoreMesh(core_axis_name="c", subcore_axis_name="s")
    run = functools.partial(
        pl.kernel,
        mesh=mesh,
        out_type=[
            jax.ShapeDtypeStruct((_N_WORKERS, 16 * _GROUP_ROWS), jnp.float32),
            jax.ShapeDtypeStruct((_N_WORKERS, 16 * _GROUP_ROWS), jnp.int32),
        ],
        scratch_types=[
            pltpu.VMEM((_GROUP_ROWS, _CHUNK_COLS), jnp.float32),
            pltpu.VMEM((_GROUP_ROWS, _CHUNK_COLS), jnp.float32),
            pltpu.VMEM((_GROUP_ROWS, 256), jnp.float32),
            pltpu.VMEM((_GROUP_ROWS, 256), jnp.int32),
            pltpu.VMEM((16 * _GROUP_ROWS,), jnp.float32),
            pltpu.VMEM((16 * _GROUP_ROWS,), jnp.int32),
            pltpu.SemaphoreType.DMA,
            pltpu.SemaphoreType.DMA,
        ],
        compiler_params=pltpu.CompilerParams(
            needs_layout_passes=False, use_tc_tiling_on_sc=True),
    )(_sc_topk_body)
    vals, idx = run(scores)
    vals = vals.reshape(_N_ROWS // _GROUP_ROWS, 2, _GROUP_ROWS, 16)
    idx = idx.reshape(_N_ROWS // _GROUP_ROWS, 2, _GROUP_ROWS, 16)
    va = vals[:, 0].reshape(_N_ROWS, 16)
    vb = vals[:, 1].reshape(_N_ROWS, 16)
    ia = idx[:, 0].reshape(_N_ROWS, 16)
    ib = idx[:, 1].reshape(_N_ROWS, 16)
    return va, ia, vb, ib


def _lex_ge(av, ai, bv, bi):
    return (av > bv) | ((av == bv) & (ai < bi))


def _tc_write_kernel(va_ref, ia_ref, vb_ref, ib_ref, o_ref):
    r, c = o_ref.shape
    a = [(va_ref[:, k:k + 1], ia_ref[:, k:k + 1]) for k in range(3)]
    b = [(vb_ref[:, k:k + 1], ib_ref[:, k:k + 1]) for k in range(3)]

    def sel(cond, x, y):
        return (jnp.where(cond, x[0], y[0]), jnp.where(cond, x[1], y[1]))

    out_vi = []
    ah, am, al = a
    bh, bm, bl = b
    for _k in range(3):
        ge = _lex_ge(ah[0], ah[1], bh[0], bh[1])
        out_vi.append(sel(ge, ah, bh))
        ah, am, al = sel(ge, am, ah), sel(ge, al, am), al
        bh, bm, bl = sel(~ge, bm, bh), sel(~ge, bl, bm), bl

    denom = out_vi[0][0] + out_vi[1][0] + out_vi[2][0]
    inv = jnp.float32(1.0) / jnp.maximum(denom, jnp.float32(1e-12))
    iota = lax.broadcasted_iota(jnp.int32, (r, c), 1)
    out = jnp.zeros((r, c), jnp.float32)
    for k in range(3):
        vk, ik = out_vi[k]
        out = jnp.where(iota == ik, vk * inv, out)
    o_ref[...] = out


def kernel(scores):
    n, c = scores.shape
    va, ia, vb, ib = _sc_topk(scores)
    rows_per_block = _GROUP_ROWS
    grid = n // rows_per_block
    spec16 = pl.BlockSpec((rows_per_block, 16), lambda i: (i, 0))
    return pl.pallas_call(
        _tc_write_kernel,
        grid=(grid,),
        in_specs=[spec16, spec16, spec16, spec16],
        out_specs=pl.BlockSpec((rows_per_block, c), lambda i: (i, 0)),
        out_shape=jax.ShapeDtypeStruct((n, c), scores.dtype),
    )(va, ia, vb, ib)


# speedup vs baseline: 1.0183x; 1.0183x over previous
import functools

import jax
import jax.numpy as jnp
from jax import lax
from jax.experimental import pallas as pl
from jax.experimental.pallas import tpu as pltpu
from jax.experimental.pallas import tpu_sc as plsc

_N_ROWS = 128
_N_COLS = 32768
_N_WORKERS = 32
_GROUP_ROWS = 8
_HALF_COLS = _N_COLS // 2
_CHUNK_COLS = 2048
_N_CHUNKS = _HALF_COLS // _CHUNK_COLS
_VECS_PER_BLOCK = 16
_BLOCK_COLS = 16 * _VECS_PER_BLOCK
_BLOCKS_PER_CHUNK = _CHUNK_COLS // _BLOCK_COLS


def _insert(v, n, t1, t2, t3, x1, x2, x3):
    c1 = v > t1
    c2 = v > t2
    c3 = v > t3
    nt1 = jnp.where(c1, v, t1)
    nx1 = jnp.where(c1, n, x1)
    nt2 = jnp.where(c1, t1, jnp.where(c2, v, t2))
    nx2 = jnp.where(c1, x1, jnp.where(c2, n, x2))
    nt3 = jnp.where(c2, t2, jnp.where(c3, v, t3))
    nx3 = jnp.where(c2, x2, jnp.where(c3, n, x3))
    return nt1, nt2, nt3, nx1, nx2, nx3


def _scan_chunk_row(buf, stv, sti, r, chunk_vec0):
    sa = _load_state(stv, sti, r, 0)
    sb = _load_state(stv, sti, r, 48)
    na0 = jnp.full((16,), 0, jnp.int32) + chunk_vec0
    nb0 = na0 + 1

    def body(i, carry):
        a = carry[0:6]
        b = carry[6:12]
        na = carry[12]
        nb = carry[13]
        for u in range(2):
            base = (i * 4 + 2 * u) * 16
            va = buf[r, pl.ds(base, 16)]
            vb = buf[r, pl.ds(base + 16, 16)]
            a = _insert(va, na, *a)
            b = _insert(vb, nb, *b)
            na = na + 2
            nb = nb + 2
        return a + b + (na, nb)

    out = lax.fori_loop(0, _CHUNK_COLS // (16 * 4), body,
                        sa + sb + (na0, nb0))
    _store_state(stv, sti, r, out[0:6], 0)
    _store_state(stv, sti, r, out[6:12], 48)


def _load_state(stv, sti, r, off):
    return (stv[r, pl.ds(off, 16)], stv[r, pl.ds(off + 16, 16)],
            stv[r, pl.ds(off + 32, 16)],
            sti[r, pl.ds(off, 16)], sti[r, pl.ds(off + 16, 16)],
            sti[r, pl.ds(off + 32, 16)])


def _store_state(stv, sti, r, s, off):
    for k in range(3):
        stv[r, pl.ds(off + 16 * k, 16)] = s[k]
        sti[r, pl.ds(off + 16 * k, 16)] = s[3 + k]


def _sc_topk_body(scores_hbm, vals_hbm, idx_hbm, buf_a, buf_b, stv, sti,
                  rv, ri, sem_a, sem_b):
    wid = lax.axis_index("s") * 2 + lax.axis_index("c")
    g = wid // 2
    h = wid % 2
    row0 = g * _GROUP_ROWS
    col0 = h * _HALF_COLS

    neg = jnp.full((16,), -jnp.inf, jnp.float32)
    zero = jnp.zeros((16,), jnp.int32)
    for r in range(_GROUP_ROWS):
        for j in range(2):
            _store_state(stv, sti, r, (neg, neg, neg, zero, zero, zero), 48 * j)

    def chunk_src(c):
        start = pl.multiple_of(col0 + c * _CHUNK_COLS, _CHUNK_COLS)
        return scores_hbm.at[pl.ds(row0, _GROUP_ROWS),
                             pl.ds(start, _CHUNK_COLS)]

    def scan_buf(buf, c):
        chunk_vec0 = c * (_CHUNK_COLS // 16)
        for r in range(_GROUP_ROWS):
            _scan_chunk_row(buf, stv, sti, r, chunk_vec0)

    last = _N_CHUNKS - 1
    pltpu.async_copy(chunk_src(0), buf_a, sem_a).wait()

    def pair(p, carry):
        c = p * 2
        cp_b = pltpu.async_copy(chunk_src(jnp.minimum(c + 1, last)), buf_b, sem_b)
        scan_buf(buf_a, c)
        cp_b.wait()
        cp_a = pltpu.async_copy(chunk_src(jnp.minimum(c + 2, last)), buf_a, sem_a)
        scan_buf(buf_b, c + 1)
        cp_a.wait()
        return carry

    lax.fori_loop(0, _N_CHUNKS // 2, pair, 0)

    lane = lax.broadcasted_iota(jnp.int32, (16,), 0)
    big = 1 << 30

    def lex(av, an, bv, bn):
        return (av > bv) | ((av == bv) & (an < bn))

    def psel(cond, x, y):
        return (jnp.where(cond, x[0], y[0]), jnp.where(cond, x[1], y[1]))

    def pair_merge(a, b):
        ah, am, al = (a[0], a[3]), (a[1], a[4]), (a[2], a[5])
        bh, bm, bl = (b[0], b[3]), (b[1], b[4]), (b[2], b[5])
        merged = []
        for _k in range(3):
            ge = lex(ah[0], ah[1], bh[0], bh[1])
            merged.append(psel(ge, ah, bh))
            ah, am, al = psel(ge, am, ah), psel(ge, al, am), al
            bh, bm, bl = psel(~ge, bm, bh), psel(~ge, bl, bm), bl
        return (merged[0][0], merged[1][0], merged[2][0],
                merged[0][1], merged[1][1], merged[2][1])

    for r in range(_GROUP_ROWS):
        s0 = _load_state(stv, sti, r, 0)
        s1 = _load_state(stv, sti, r, 48)
        t1, t2, t3, x1, x2, x3 = pair_merge(s0, s1)
        g1 = x1 * 16 + lane + col0
        g2 = x2 * 16 + lane + col0
        g3 = x3 * 16 + lane + col0
        ms = []
        gs = []
        for _round in range(3):
            mx = jnp.max(t1)
            gi = jnp.min(jnp.where(t1 == mx, g1, big))
            win = g1 == gi
            ms.append(mx)
            gs.append(gi)
            t1 = jnp.where(win, t2, t1)
            g1 = jnp.where(win, g2, g1)
            t2 = jnp.where(win, t3, t2)
            g2 = jnp.where(win, g3, g2)
            t3 = jnp.where(win, -jnp.inf, t3)
        l0 = lane == 0
        l1 = lane == 1
        l2 = lane == 2
        valv = jnp.where(l0, ms[0],
                         jnp.where(l1, ms[1],
                                   jnp.where(l2, ms[2], jnp.float32(0.0))))
        idxv = jnp.where(l0, gs[0],
                         jnp.where(l1, gs[1], jnp.where(l2, gs[2], 0)))
        rv[pl.ds(16 * r, 16)] = valv
        ri[pl.ds(16 * r, 16)] = idxv
    pltpu.sync_copy(rv, vals_hbm.at[wid])
    pltpu.sync_copy(ri, idx_hbm.at[wid])


def _sc_topk(scores):
    mesh = plsc.VectorSubcoreMesh(core_axis_name="c", subcore_axis_name="s")
    run = functools.partial(
        pl.kernel,
        mesh=mesh,
        out_type=[
            jax.ShapeDtypeStruct((_N_WORKERS, 16 * _GROUP_ROWS), jnp.float32),
            jax.ShapeDtypeStruct((_N_WORKERS, 16 * _GROUP_ROWS), jnp.int32),
        ],
        scratch_types=[
            pltpu.VMEM((_GROUP_ROWS, _CHUNK_COLS), jnp.float32),
            pltpu.VMEM((_GROUP_ROWS, _CHUNK_COLS), jnp.float32),
            pltpu.VMEM((_GROUP_ROWS, 256), jnp.float32),
            pltpu.VMEM((_GROUP_ROWS, 256), jnp.int32),
            pltpu.VMEM((16 * _GROUP_ROWS,), jnp.float32),
            pltpu.VMEM((16 * _GROUP_ROWS,), jnp.int32),
            pltpu.SemaphoreType.DMA,
            pltpu.SemaphoreType.DMA,
        ],
        compiler_params=pltpu.CompilerParams(
            needs_layout_passes=False, use_tc_tiling_on_sc=True),
    )(_sc_topk_body)
    vals, idx = run(scores)
    vals = vals.reshape(_N_ROWS // _GROUP_ROWS, 2, _GROUP_ROWS, 16)
    idx = idx.reshape(_N_ROWS // _GROUP_ROWS, 2, _GROUP_ROWS, 16)
    va = vals[:, 0].reshape(_N_ROWS, 16)
    vb = vals[:, 1].reshape(_N_ROWS, 16)
    ia = idx[:, 0].reshape(_N_ROWS, 16)
    ib = idx[:, 1].reshape(_N_ROWS, 16)
    return va, ia, vb, ib


def _lex_ge(av, ai, bv, bi):
    return (av > bv) | ((av == bv) & (ai < bi))


def _tc_write_kernel(va_ref, ia_ref, vb_ref, ib_ref, o_ref):
    r, c = o_ref.shape
    a = [(va_ref[:, k:k + 1], ia_ref[:, k:k + 1]) for k in range(3)]
    b = [(vb_ref[:, k:k + 1], ib_ref[:, k:k + 1]) for k in range(3)]

    def sel(cond, x, y):
        return (jnp.where(cond, x[0], y[0]), jnp.where(cond, x[1], y[1]))

    out_vi = []
    ah, am, al = a
    bh, bm, bl = b
    for _k in range(3):
        ge = _lex_ge(ah[0], ah[1], bh[0], bh[1])
        out_vi.append(sel(ge, ah, bh))
        ah, am, al = sel(ge, am, ah), sel(ge, al, am), al
        bh, bm, bl = sel(~ge, bm, bh), sel(~ge, bl, bm), bl

    denom = out_vi[0][0] + out_vi[1][0] + out_vi[2][0]
    inv = jnp.float32(1.0) / jnp.maximum(denom, jnp.float32(1e-12))
    iota = lax.broadcasted_iota(jnp.int32, (r, c), 1)
    out = jnp.zeros((r, c), jnp.float32)
    for k in range(3):
        vk, ik = out_vi[k]
        out = jnp.where(iota == ik, vk * inv, out)
    o_ref[...] = out


def kernel(scores):
    n, c = scores.shape
    va, ia, vb, ib = _sc_topk(scores)
    rows_per_block = _GROUP_ROWS
    grid = n // rows_per_block
    spec16 = pl.BlockSpec((rows_per_block, 16), lambda i: (i, 0))
    return pl.pallas_call(
        _tc_write_kernel,
        grid=(grid,),
        in_specs=[spec16, spec16, spec16, spec16],
        out_specs=pl.BlockSpec((rows_per_block, c), lambda i: (i, 0)),
        out_shape=jax.ShapeDtypeStruct((n, c), scores.dtype),
    )(va, ia, vb, ib)
